# baseline (device time: 93346 ns/iter reference)
import jax
import jax.numpy as jnp
from jax import lax
from jax.experimental import pallas as pl
from jax.experimental.pallas import tpu as pltpu

P = 32
M, N = 2048, 1024
R = M // 2

X_MASKS = (1,)
Y_MASKS = (3, 4, 7)
Z_MASKS = (8, 16, 24)
ALL_MASKS = (1, 3, 4, 7, 8, 16, 24)

KINDS = {"x": (X_MASKS, 2), "y": (Y_MASKS, 4), "z": (Z_MASKS, 4)}
ORDER_A = ("x", "y", "z")
ORDER_B = ("y", "z", "x")


def _coord(dev, kind):
    if kind == "x":
        return jnp.bitwise_and(jnp.bitwise_xor(dev, jnp.right_shift(dev, 1)), 1)
    if kind == "y":
        return jnp.bitwise_and(jnp.right_shift(dev, 1), 3)
    return jnp.bitwise_and(jnp.right_shift(dev, 3), 3)


def kernel(x):
    rs_layout = {}
    comm_rows = 0
    rs_sems = 0
    for k in range(3):
        for s, order in ((0, ORDER_A), (1, ORDER_B)):
            kind = order[k]
            masks, parts = KINDS[kind]
            active = R
            for kk in range(k):
                active //= KINDS[order[kk]][1]
            part = active // parts
            rs_layout[(s, k)] = (comm_rows, rs_sems, active)
            comm_rows += part * len(masks)
            rs_sems += len(masks)
    ag_layout = {}
    ag_sems = 0
    for j in range(3):
        for s, order in ((0, ORDER_A), (1, ORDER_B)):
            kind = order[2 - j]
            ag_layout[(s, j)] = ag_sems
            ag_sems += len(KINDS[kind][0])

    def body(x_ref, out_ref, comm_ref, rs_send, rs_recv, ag_send, ag_recv):
        me = lax.axis_index("i")

        barrier_sem = pltpu.get_barrier_semaphore()
        for v in ALL_MASKS:
            pl.semaphore_signal(
                barrier_sem, inc=1,
                device_id=(jnp.bitwise_xor(me, v),),
                device_id_type=pl.DeviceIdType.MESH,
            )
        pl.semaphore_wait(barrier_sem, len(ALL_MASKS))

        out_ref[...] = x_ref[...].astype(jnp.bfloat16)

        off = [jnp.int32(0), jnp.int32(R)]
        for k in range(3):
            pend = []
            for s, order in ((0, ORDER_A), (1, ORDER_B)):
                kind = order[k]
                masks, parts = KINDS[kind]
                slot_base, sem_base, active = rs_layout[(s, k)]
                part = active // parts
                keep_off = off[s] + _coord(me, kind) * part
                rdmas = []
                for d, m in enumerate(masks):
                    partner = jnp.bitwise_xor(me, m)
                    send_off = off[s] + _coord(partner, kind) * part
                    rdma = pltpu.make_async_remote_copy(
                        src_ref=out_ref.at[pl.ds(send_off, part)],
                        dst_ref=comm_ref.at[pl.ds(slot_base + d * part, part)],
                        send_sem=rs_send.at[sem_base + d],
                        recv_sem=rs_recv.at[sem_base + d],
                        device_id=(partner,),
                        device_id_type=pl.DeviceIdType.MESH,
                    )
                    rdma.start()
                    rdmas.append(rdma)
                pend.append((s, rdmas, keep_off, part, slot_base, len(masks)))
            for s, rdmas, keep_off, part, slot_base, nm in pend:
                for rdma in rdmas:
                    rdma.wait()
                acc = out_ref[pl.ds(keep_off, part)].astype(jnp.float32)
                for d in range(nm):
                    acc = acc + comm_ref[
                        pl.ds(slot_base + d * part, part)
                    ].astype(jnp.float32)
                out_ref[pl.ds(keep_off, part)] = acc.astype(jnp.bfloat16)
                off[s] = keep_off


        cur = [R // 32, R // 32]
        for j in range(3):
            pend = []
            for s, order in ((0, ORDER_A), (1, ORDER_B)):
                kind = order[2 - j]
                masks, parts = KINDS[kind]
                sem_base = ag_layout[(s, j)]
                rdmas = []
                for d, m in enumerate(masks):
                    partner = jnp.bitwise_xor(me, m)
                    rdma = pltpu.make_async_remote_copy(
                        src_ref=out_ref.at[pl.ds(off[s], cur[s])],
                        dst_ref=out_ref.at[pl.ds(off[s], cur[s])],
                        send_sem=ag_send.at[sem_base + d],
                        recv_sem=ag_recv.at[sem_base + d],
                        device_id=(partner,),
                        device_id_type=pl.DeviceIdType.MESH,
                    )
                    rdma.start()
                    rdmas.append(rdma)
                pend.append((s, kind, parts, rdmas))
            for s, kind, parts, rdmas in pend:
                for rdma in rdmas:
                    rdma.wait()
                off[s] = off[s] - _coord(me, kind) * cur[s]
                cur[s] = cur[s] * parts

    return pl.pallas_call(
        body,
        out_shape=jax.ShapeDtypeStruct((M, N), jnp.bfloat16),
        in_specs=[pl.BlockSpec(memory_space=pltpu.VMEM)],
        out_specs=pl.BlockSpec(memory_space=pltpu.VMEM),
        scratch_shapes=[
            pltpu.VMEM((comm_rows, N), jnp.bfloat16),
            pltpu.SemaphoreType.DMA((rs_sems,)),
            pltpu.SemaphoreType.DMA((rs_sems,)),
            pltpu.SemaphoreType.DMA((ag_sems,)),
            pltpu.SemaphoreType.DMA((ag_sems,)),
        ],
        compiler_params=pltpu.CompilerParams(collective_id=0),
    )(x)


# device time: 78293 ns/iter; 1.1923x vs baseline; 1.1923x over previous
import jax
import jax.numpy as jnp
from jax import lax
from jax.experimental import pallas as pl
from jax.experimental.pallas import tpu as pltpu

P = 32
MASKS = (1, 3, 4, 8, 16)
ORDERS = ((1, 8, 3, 4, 16), (8, 3, 1, 16, 4))


def _keep_bit(me, v):
    if v == 1:
        return jnp.bitwise_and(jnp.bitwise_xor(me, jnp.right_shift(me, 1)), 1)
    if v == 3:
        return jnp.bitwise_and(jnp.right_shift(me, 1), 1)
    shift = {4: 2, 8: 3, 16: 4}[v]
    return jnp.bitwise_and(jnp.right_shift(me, shift), 1)


def kernel(x):
    M, N = x.shape
    R = M // 2
    sizes = [R >> (k + 1) for k in range(5)]
    stream_rows = sum(sizes)

    def slot(s, k):
        return s * stream_rows + sum(sizes[:k])

    def body(x_ref, out_ref, comm_ref, rs_send, rs_recv, ag_send, ag_recv):
        me = lax.axis_index("i")

        barrier_sem = pltpu.get_barrier_semaphore()
        for v in MASKS:
            pl.semaphore_signal(
                barrier_sem, inc=1,
                device_id=(jnp.bitwise_xor(me, v),),
                device_id_type=pl.DeviceIdType.MESH,
            )
        pl.semaphore_wait(barrier_sem, len(MASKS))

        out_ref[...] = x_ref[...].astype(jnp.bfloat16)

        off = [jnp.int32(0), jnp.int32(R)]
        for k in range(5):
            sz = sizes[k]
            rdmas = []
            keep = []
            for s in (0, 1):
                v = ORDERS[s][k]
                partner = jnp.bitwise_xor(me, v)
                mybit = _keep_bit(me, v)
                send_off = off[s] + (1 - mybit) * sz
                keep.append(off[s] + mybit * sz)
                rdma = pltpu.make_async_remote_copy(
                    src_ref=out_ref.at[pl.ds(send_off, sz)],
                    dst_ref=comm_ref.at[pl.ds(slot(s, k), sz)],
                    send_sem=rs_send.at[s * 5 + k],
                    recv_sem=rs_recv.at[s * 5 + k],
                    device_id=(partner,),
                    device_id_type=pl.DeviceIdType.MESH,
                )
                rdma.start()
                rdmas.append(rdma)
            for s in (0, 1):
                rdmas[s].wait()
                out_ref[pl.ds(keep[s], sz)] = (
                    out_ref[pl.ds(keep[s], sz)].astype(jnp.float32)
                    + comm_ref[pl.ds(slot(s, k), sz)].astype(jnp.float32)
                ).astype(jnp.bfloat16)
                off[s] = keep[s]


        for j in range(5):
            cur = sizes[4] << j
            rdmas = []
            bits = []
            for s in (0, 1):
                v = ORDERS[s][4 - j]
                partner = jnp.bitwise_xor(me, v)
                bits.append(_keep_bit(me, v))
                rdma = pltpu.make_async_remote_copy(
                    src_ref=out_ref.at[pl.ds(off[s], cur)],
                    dst_ref=out_ref.at[pl.ds(off[s], cur)],
                    send_sem=ag_send.at[s * 5 + j],
                    recv_sem=ag_recv.at[s * 5 + j],
                    device_id=(partner,),
                    device_id_type=pl.DeviceIdType.MESH,
                )
                rdma.start()
                rdmas.append(rdma)
            for s in (0, 1):
                rdmas[s].wait()
                off[s] = off[s] - bits[s] * cur

    return pl.pallas_call(
        body,
        out_shape=jax.ShapeDtypeStruct((M, N), jnp.bfloat16),
        in_specs=[pl.BlockSpec(memory_space=pltpu.VMEM)],
        out_specs=pl.BlockSpec(memory_space=pltpu.VMEM),
        scratch_shapes=[
            pltpu.VMEM((2 * stream_rows, N), jnp.bfloat16),
            pltpu.SemaphoreType.DMA((10,)),
            pltpu.SemaphoreType.DMA((10,)),
            pltpu.SemaphoreType.DMA((10,)),
            pltpu.SemaphoreType.DMA((10,)),
        ],
        compiler_params=pltpu.CompilerParams(collective_id=0),
    )(x)


# device time: 66852 ns/iter; 1.3963x vs baseline; 1.1711x over previous
import jax
import jax.numpy as jnp
from jax import lax
from jax.experimental import pallas as pl
from jax.experimental.pallas import tpu as pltpu

P = 32
MASKS = (1, 3, 4, 8, 16)
ORDERS = ((1, 8, 3, 4, 16), (8, 3, 1, 16, 4))

AG_EX = [(i, j) for j in range(5) for i in range(-1, j)]


def _keep_bit(me, v):
    if v == 1:
        return jnp.bitwise_and(jnp.bitwise_xor(me, jnp.right_shift(me, 1)), 1)
    if v == 3:
        return jnp.bitwise_and(jnp.right_shift(me, 1), 1)
    shift = {4: 2, 8: 3, 16: 4}[v]
    return jnp.bitwise_and(jnp.right_shift(me, shift), 1)


def kernel(x):
    M, N = x.shape
    R = M // 2
    sizes = [R >> (k + 1) for k in range(5)]
    stream_rows = sum(sizes)
    base = sizes[4]

    def slot(s, k):
        return s * stream_rows + sum(sizes[:k])

    def ag_sem(s, i, j):
        return s * len(AG_EX) + AG_EX.index((i, j))

    def body(x_ref, out_ref, comm_ref, rs_send, rs_recv, ag_send, ag_recv):
        me = lax.axis_index("i")

        barrier_sem = pltpu.get_barrier_semaphore()
        for v in MASKS:
            pl.semaphore_signal(
                barrier_sem, inc=1,
                device_id=(jnp.bitwise_xor(me, v),),
                device_id_type=pl.DeviceIdType.MESH,
            )
        pl.semaphore_wait(barrier_sem, len(MASKS))

        def start_rs(s, k, src_off):
            v = ORDERS[s][k]
            rdma = pltpu.make_async_remote_copy(
                src_ref=out_ref.at[pl.ds(src_off, sizes[k])],
                dst_ref=comm_ref.at[pl.ds(slot(s, k), sizes[k])],
                send_sem=rs_send.at[s * 5 + k],
                recv_sem=rs_recv.at[s * 5 + k],
                device_id=(jnp.bitwise_xor(me, v),),
                device_id_type=pl.DeviceIdType.MESH,
            )
            rdma.start()
            return rdma

        rdmas = [None, None]
        off = [None, None]
        for s in (0, 1):
            sbase = s * R
            out_ref[pl.ds(sbase, R)] = x_ref[pl.ds(sbase, R)].astype(
                jnp.bfloat16
            )
            bit = _keep_bit(me, ORDERS[s][0])
            off[s] = sbase + bit * sizes[0]
            rdmas[s] = start_rs(s, 0, sbase + (1 - bit) * sizes[0])

        for k in range(5):
            sz = sizes[k]
            late = []
            for s in (0, 1):
                rdmas[s].wait()
                if k < 4:
                    szn = sizes[k + 1]
                    bitn = _keep_bit(me, ORDERS[s][k + 1])
                    send_off = off[s] + (1 - bitn) * szn
                    keep_off = off[s] + bitn * szn
                    csub = slot(s, k) + (send_off - off[s])
                    out_ref[pl.ds(send_off, szn)] = (
                        out_ref[pl.ds(send_off, szn)].astype(jnp.float32)
                        + comm_ref[pl.ds(csub, szn)].astype(jnp.float32)
                    ).astype(jnp.bfloat16)
                    rdmas[s] = start_rs(s, k + 1, send_off)
                    late.append((s, keep_off, szn, slot(s, k) + (keep_off - off[s])))
                    off[s] = keep_off
                else:
                    out_ref[pl.ds(off[s], sz)] = (
                        out_ref[pl.ds(off[s], sz)].astype(jnp.float32)
                        + comm_ref[pl.ds(slot(s, k), sz)].astype(jnp.float32)
                    ).astype(jnp.bfloat16)
            for s, keep_off, szn, csub in late:
                out_ref[pl.ds(keep_off, szn)] = (
                    out_ref[pl.ds(keep_off, szn)].astype(jnp.float32)
                    + comm_ref[pl.ds(csub, szn)].astype(jnp.float32)
                ).astype(jnp.bfloat16)


        pieces = [{-1: (off[s], base)} for s in (0, 1)]
        bits_ag = [[_keep_bit(me, ORDERS[s][4 - j]) for j in range(5)]
                   for s in (0, 1)]

        def ag_start(s, i, j, piece_off, piece_sz):
            v = ORDERS[s][4 - j]
            rdma = pltpu.make_async_remote_copy(
                src_ref=out_ref.at[pl.ds(piece_off, piece_sz)],
                dst_ref=out_ref.at[pl.ds(piece_off, piece_sz)],
                send_sem=ag_send.at[ag_sem(s, i, j)],
                recv_sem=ag_recv.at[ag_sem(s, i, j)],
                device_id=(jnp.bitwise_xor(me, v),),
                device_id_type=pl.DeviceIdType.MESH,
            )
            rdma.start()
            return rdma

        started = {}
        for s in (0, 1):
            for j in range(5):
                po, psz = pieces[s][-1]
                started[(s, -1, j)] = ag_start(s, -1, j, po, psz)

        cur_off = [off[0], off[1]]
        for i in range(5):
            curi = base << i
            for s in (0, 1):
                for ii in range(-1, i):
                    started[(s, ii, i)].wait_recv()
                p_off = cur_off[s] + (1 - 2 * bits_ag[s][i]) * curi
                pieces[s][i] = (p_off, curi)
                cur_off[s] = cur_off[s] - bits_ag[s][i] * curi
                for j in range(i + 1, 5):
                    started[(s, i, j)] = ag_start(s, i, j, p_off, curi)
        for key, rdma in started.items():
            rdma.wait_send()

    n_ag = 2 * len(AG_EX)
    return pl.pallas_call(
        body,
        out_shape=jax.ShapeDtypeStruct((M, N), jnp.bfloat16),
        in_specs=[pl.BlockSpec(memory_space=pltpu.VMEM)],
        out_specs=pl.BlockSpec(memory_space=pltpu.VMEM),
        scratch_shapes=[
            pltpu.VMEM((2 * stream_rows, N), jnp.bfloat16),
            pltpu.SemaphoreType.DMA((10,)),
            pltpu.SemaphoreType.DMA((10,)),
            pltpu.SemaphoreType.DMA((n_ag,)),
            pltpu.SemaphoreType.DMA((n_ag,)),
        ],
        compiler_params=pltpu.CompilerParams(collective_id=0),
    )(x)


# device time: 60200 ns/iter; 1.5506x vs baseline; 1.1105x over previous
import jax
import jax.numpy as jnp
from jax import lax
from jax.experimental import pallas as pl
from jax.experimental.pallas import tpu as pltpu

P = 32
MASKS = (1, 3, 4, 8, 16)
ORDERS = ((1, 8, 3, 4, 16), (8, 3, 1, 16, 4))

AG_EX = [(i, j) for j in range(5) for i in range(-1, j)]


def _keep_bit(me, v):
    if v == 1:
        return jnp.bitwise_and(jnp.bitwise_xor(me, jnp.right_shift(me, 1)), 1)
    if v == 3:
        return jnp.bitwise_and(jnp.right_shift(me, 1), 1)
    shift = {4: 2, 8: 3, 16: 4}[v]
    return jnp.bitwise_and(jnp.right_shift(me, shift), 1)


def kernel(x):
    M, N = x.shape
    R = M // 2
    sizes = [R >> (k + 1) for k in range(5)]
    stream_rows = sum(sizes)
    base = sizes[4]

    def slot(s, k):
        return s * stream_rows + sum(sizes[:k])

    def ag_sem(s, i, j):
        return s * len(AG_EX) + AG_EX.index((i, j))

    def body(x_ref, out_ref, comm_ref, rs_send, rs_recv, ag_send, ag_recv):
        me = lax.axis_index("i")

        barrier_sem = pltpu.get_barrier_semaphore()
        for v in MASKS:
            pl.semaphore_signal(
                barrier_sem, inc=1,
                device_id=(jnp.bitwise_xor(me, v),),
                device_id_type=pl.DeviceIdType.MESH,
            )
        pl.semaphore_wait(barrier_sem, len(MASKS))

        def _rs_copy(s, k, part, src_off, rows, dst_sub):
            v = ORDERS[s][k]
            rdma = pltpu.make_async_remote_copy(
                src_ref=out_ref.at[pl.ds(src_off, rows)],
                dst_ref=comm_ref.at[pl.ds(slot(s, k) + dst_sub, rows)],
                send_sem=rs_send.at[(s * 5 + k) * 2 + part],
                recv_sem=rs_recv.at[(s * 5 + k) * 2 + part],
                device_id=(jnp.bitwise_xor(me, v),),
                device_id_type=pl.DeviceIdType.MESH,
            )
            rdma.start()
            return rdma

        def start_rs(s, k, src_off):
            if k == 4:
                return (_rs_copy(s, k, 0, src_off, sizes[k], 0),)
            partner = jnp.bitwise_xor(me, ORDERS[s][k])
            pbit = _keep_bit(partner, ORDERS[s][k + 1])
            szn = sizes[k + 1]
            sub_a = (1 - pbit) * szn
            sub_b = pbit * szn
            return (
                _rs_copy(s, k, 0, src_off + sub_a, szn, sub_a),
                _rs_copy(s, k, 1, src_off + sub_b, szn, sub_b),
            )

        rdmas = [None, None]
        off = [None, None]
        for s in (0, 1):
            sbase = s * R
            out_ref[pl.ds(sbase, R)] = x_ref[pl.ds(sbase, R)].astype(
                jnp.bfloat16
            )
            bit = _keep_bit(me, ORDERS[s][0])
            off[s] = sbase + bit * sizes[0]
            rdmas[s] = start_rs(s, 0, sbase + (1 - bit) * sizes[0])

        for k in range(5):
            sz = sizes[k]
            late = []
            for s in (0, 1):
                if k < 4:
                    szn = sizes[k + 1]
                    bitn = _keep_bit(me, ORDERS[s][k + 1])
                    send_off = off[s] + (1 - bitn) * szn
                    keep_off = off[s] + bitn * szn
                    rdmas[s][0].wait()
                    csub = slot(s, k) + (send_off - off[s])
                    out_ref[pl.ds(send_off, szn)] = (
                        out_ref[pl.ds(send_off, szn)].astype(jnp.float32)
                        + comm_ref[pl.ds(csub, szn)].astype(jnp.float32)
                    ).astype(jnp.bfloat16)
                    nxt = start_rs(s, k + 1, send_off)
                    late.append(
                        (s, rdmas[s][1], keep_off, szn,
                         slot(s, k) + (keep_off - off[s]))
                    )
                    rdmas[s] = nxt
                    off[s] = keep_off
                else:
                    rdmas[s][0].wait()
                    out_ref[pl.ds(off[s], sz)] = (
                        out_ref[pl.ds(off[s], sz)].astype(jnp.float32)
                        + comm_ref[pl.ds(slot(s, k), sz)].astype(jnp.float32)
                    ).astype(jnp.bfloat16)
            for s, rdma_b, keep_off, szn, csub in late:
                rdma_b.wait()
                out_ref[pl.ds(keep_off, szn)] = (
                    out_ref[pl.ds(keep_off, szn)].astype(jnp.float32)
                    + comm_ref[pl.ds(csub, szn)].astype(jnp.float32)
                ).astype(jnp.bfloat16)


        pieces = [{-1: (off[s], base)} for s in (0, 1)]
        bits_ag = [[_keep_bit(me, ORDERS[s][4 - j]) for j in range(5)]
                   for s in (0, 1)]

        def ag_start(s, i, j, piece_off, piece_sz):
            v = ORDERS[s][4 - j]
            rdma = pltpu.make_async_remote_copy(
                src_ref=out_ref.at[pl.ds(piece_off, piece_sz)],
                dst_ref=out_ref.at[pl.ds(piece_off, piece_sz)],
                send_sem=ag_send.at[ag_sem(s, i, j)],
                recv_sem=ag_recv.at[ag_sem(s, i, j)],
                device_id=(jnp.bitwise_xor(me, v),),
                device_id_type=pl.DeviceIdType.MESH,
            )
            rdma.start()
            return rdma

        started = {}
        for s in (0, 1):
            for j in range(5):
                po, psz = pieces[s][-1]
                started[(s, -1, j)] = ag_start(s, -1, j, po, psz)

        cur_off = [off[0], off[1]]
        for i in range(5):
            curi = base << i
            for s in (0, 1):
                for ii in range(-1, i):
                    started[(s, ii, i)].wait_recv()
                p_off = cur_off[s] + (1 - 2 * bits_ag[s][i]) * curi
                pieces[s][i] = (p_off, curi)
                cur_off[s] = cur_off[s] - bits_ag[s][i] * curi
                for j in range(i + 1, 5):
                    started[(s, i, j)] = ag_start(s, i, j, p_off, curi)
        for key, rdma in started.items():
            rdma.wait_send()

    n_ag = 2 * len(AG_EX)
    return pl.pallas_call(
        body,
        out_shape=jax.ShapeDtypeStruct((M, N), jnp.bfloat16),
        in_specs=[pl.BlockSpec(memory_space=pltpu.VMEM)],
        out_specs=pl.BlockSpec(memory_space=pltpu.VMEM),
        scratch_shapes=[
            pltpu.VMEM((2 * stream_rows, N), jnp.bfloat16),
            pltpu.SemaphoreType.DMA((20,)),
            pltpu.SemaphoreType.DMA((20,)),
            pltpu.SemaphoreType.DMA((n_ag,)),
            pltpu.SemaphoreType.DMA((n_ag,)),
        ],
        compiler_params=pltpu.CompilerParams(collective_id=0),
    )(x)


# device time: 53901 ns/iter; 1.7318x vs baseline; 1.1169x over previous
import jax
import jax.numpy as jnp
from jax import lax
from jax.experimental import pallas as pl
from jax.experimental.pallas import tpu as pltpu

P = 32
MASKS = (1, 3, 4, 8, 16)
STREAMS = (
    (0, 384, (1, 8, 3, 4, 16)),
    (384, 384, (8, 3, 1, 16, 4)),
    (768, 256, (3, 1, 16, 8, 4)),
)
NS = len(STREAMS)

AG_EX = [(i, j) for j in range(5) for i in range(-1, j)]


def _keep_bit(me, v):
    if v == 1:
        return jnp.bitwise_and(jnp.bitwise_xor(me, jnp.right_shift(me, 1)), 1)
    if v == 3:
        return jnp.bitwise_and(jnp.right_shift(me, 1), 1)
    shift = {4: 2, 8: 3, 16: 4}[v]
    return jnp.bitwise_and(jnp.right_shift(me, shift), 1)


def kernel(x):
    M, N = x.shape
    sizes = [M >> (k + 1) for k in range(5)]
    comm_rows = sum(sizes)
    base = sizes[4]

    def slot(k):
        return sum(sizes[:k])

    def ag_sem(s, i, j):
        return s * len(AG_EX) + AG_EX.index((i, j))

    def body(x_ref, out_ref, comm_ref, rs_send, rs_recv, ag_send, ag_recv):
        me = lax.axis_index("i")

        barrier_sem = pltpu.get_barrier_semaphore()
        for v in MASKS:
            pl.semaphore_signal(
                barrier_sem, inc=1,
                device_id=(jnp.bitwise_xor(me, v),),
                device_id_type=pl.DeviceIdType.MESH,
            )
        pl.semaphore_wait(barrier_sem, len(MASKS))

        def _rs_copy(s, k, part, src_off, rows, dst_sub):
            c0, cw, order = STREAMS[s]
            rdma = pltpu.make_async_remote_copy(
                src_ref=out_ref.at[pl.ds(src_off, rows), pl.ds(c0, cw)],
                dst_ref=comm_ref.at[
                    pl.ds(slot(k) + dst_sub, rows), pl.ds(c0, cw)
                ],
                send_sem=rs_send.at[(s * 5 + k) * 2 + part],
                recv_sem=rs_recv.at[(s * 5 + k) * 2 + part],
                device_id=(jnp.bitwise_xor(me, order[k]),),
                device_id_type=pl.DeviceIdType.MESH,
            )
            rdma.start()
            return rdma

        def start_rs(s, k, src_off):
            order = STREAMS[s][2]
            if k == 4:
                return (_rs_copy(s, k, 0, src_off, sizes[k], 0),)
            partner = jnp.bitwise_xor(me, order[k])
            pbit = _keep_bit(partner, order[k + 1])
            szn = sizes[k + 1]
            sub_a = (1 - pbit) * szn
            sub_b = pbit * szn
            return (
                _rs_copy(s, k, 0, src_off + sub_a, szn, sub_a),
                _rs_copy(s, k, 1, src_off + sub_b, szn, sub_b),
            )

        def _add(s, dst_off, rows, comm_off):
            c0, cw, _ = STREAMS[s]
            out_ref[pl.ds(dst_off, rows), pl.ds(c0, cw)] = (
                out_ref[pl.ds(dst_off, rows), pl.ds(c0, cw)].astype(
                    jnp.float32
                )
                + comm_ref[pl.ds(comm_off, rows), pl.ds(c0, cw)].astype(
                    jnp.float32
                )
            ).astype(jnp.bfloat16)

        rdmas = [None] * NS
        off = [None] * NS
        for s in range(NS):
            c0, cw, order = STREAMS[s]
            bit = _keep_bit(me, order[0])
            off[s] = bit * sizes[0]
            send0 = (1 - bit) * sizes[0]
            out_ref[pl.ds(send0, sizes[0]), pl.ds(c0, cw)] = x_ref[
                pl.ds(send0, sizes[0]), pl.ds(c0, cw)
            ].astype(jnp.bfloat16)
            rdmas[s] = start_rs(s, 0, send0)
        for s in range(NS):
            c0, cw, _ = STREAMS[s]
            out_ref[pl.ds(off[s], sizes[0]), pl.ds(c0, cw)] = x_ref[
                pl.ds(off[s], sizes[0]), pl.ds(c0, cw)
            ].astype(jnp.bfloat16)

        for k in range(5):
            late = []
            for s in range(NS):
                order = STREAMS[s][2]
                if k < 4:
                    szn = sizes[k + 1]
                    bitn = _keep_bit(me, order[k + 1])
                    send_off = off[s] + (1 - bitn) * szn
                    keep_off = off[s] + bitn * szn
                    rdmas[s][0].wait()
                    _add(s, send_off, szn, slot(k) + (send_off - off[s]))
                    nxt = start_rs(s, k + 1, send_off)
                    late.append(
                        (s, rdmas[s][1], keep_off, szn,
                         slot(k) + (keep_off - off[s]))
                    )
                    rdmas[s] = nxt
                    off[s] = keep_off
                else:
                    rdmas[s][0].wait()
                    _add(s, off[s], sizes[k], slot(k))
            for s, rdma_b, keep_off, szn, csub in late:
                rdma_b.wait()
                _add(s, keep_off, szn, csub)


        bits_ag = [
            [_keep_bit(me, STREAMS[s][2][4 - j]) for j in range(5)]
            for s in range(NS)
        ]

        def ag_start(s, i, j, piece_off, piece_sz):
            c0, cw, order = STREAMS[s]
            rdma = pltpu.make_async_remote_copy(
                src_ref=out_ref.at[pl.ds(piece_off, piece_sz), pl.ds(c0, cw)],
                dst_ref=out_ref.at[pl.ds(piece_off, piece_sz), pl.ds(c0, cw)],
                send_sem=ag_send.at[ag_sem(s, i, j)],
                recv_sem=ag_recv.at[ag_sem(s, i, j)],
                device_id=(jnp.bitwise_xor(me, order[4 - j]),),
                device_id_type=pl.DeviceIdType.MESH,
            )
            rdma.start()
            return rdma

        started = {}
        for s in range(NS):
            for j in range(5):
                started[(s, -1, j)] = ag_start(s, -1, j, off[s], base)

        cur_off = list(off)
        for i in range(5):
            curi = base << i
            for s in range(NS):
                for ii in range(-1, i):
                    started[(s, ii, i)].wait_recv()
                p_off = cur_off[s] + (1 - 2 * bits_ag[s][i]) * curi
                cur_off[s] = cur_off[s] - bits_ag[s][i] * curi
                for j in range(i + 1, 5):
                    started[(s, i, j)] = ag_start(s, i, j, p_off, curi)
        for key, rdma in started.items():
            rdma.wait_send()

    n_ag = NS * len(AG_EX)
    return pl.pallas_call(
        body,
        out_shape=jax.ShapeDtypeStruct((M, N), jnp.bfloat16),
        in_specs=[pl.BlockSpec(memory_space=pltpu.VMEM)],
        out_specs=pl.BlockSpec(memory_space=pltpu.VMEM),
        scratch_shapes=[
            pltpu.VMEM((comm_rows, N), jnp.bfloat16),
            pltpu.SemaphoreType.DMA((NS * 10,)),
            pltpu.SemaphoreType.DMA((NS * 10,)),
            pltpu.SemaphoreType.DMA((n_ag,)),
            pltpu.SemaphoreType.DMA((n_ag,)),
        ],
        compiler_params=pltpu.CompilerParams(collective_id=0),
    )(x)
